# baseline (speedup 1.0000x reference)
import functools

import jax
import jax.numpy as jnp
from jax import lax
from jax.experimental import pallas as pl
from jax.experimental.pallas import tpu as pltpu
from jax.experimental.pallas import tpu_sc as plsc

_D = 32
_NC = 2
_NS = 16
_NW = _NC * _NS
_CH = 128
_NBUF = 4


def _make_relayout(V):
    nvt = (V + 127) // 128
    nfull = nvt - 1 if V % 128 else nvt
    nlast = (V - nfull * 128) * _D // 128
    grp = 4
    assert nfull % grp == 0
    ng = nfull // grp
    rounds = (ng + _NW - 1) // _NW

    @functools.partial(
        pl.kernel,
        out_type=jax.ShapeDtypeStruct((V * _D // 128, 128), jnp.float32),
        mesh=plsc.VectorSubcoreMesh(core_axis_name="c", subcore_axis_name="s"),
        compiler_params=pltpu.CompilerParams(
            use_tc_tiling_on_sc=True, needs_layout_passes=False),
        scratch_types=(
            [pltpu.VMEM((2, _D, 128 * grp), jnp.float32),
             pltpu.VMEM((2, 32 * grp, 128), jnp.float32)]
            + [pltpu.SemaphoreType.DMA] * 4
        ),
    )
    def relayout_kernel(tabt_hbm, out_hbm, in_v, ob_v, *sems):
        gsems = sems[:2]
        wsems = sems[2:]
        wid = lax.axis_index("s") * _NC + lax.axis_index("c")
        lane = lax.iota(jnp.int32, 16)

        def gid_of(k):
            return k * _NW + wid

        def start_in(k, slot):
            gid = gid_of(k)
            for dt in range(_D // 8):
                pltpu.async_copy(
                    tabt_hbm.at[pl.ds(dt * 8, 8),
                                pl.ds(gid * 128 * grp, 128 * grp)],
                    in_v.at[slot, pl.ds(dt * 8, 8)], gsems[slot])

        def transp(slot):
            ob = ob_v.at[slot]

            def tstep(g, carry):
                q = g * 16 + lane
                for d in range(_D):
                    v = in_v[slot, d, pl.ds(g * 16, 16)]
                    plsc.store_scatter(
                        ob, [q // 4, lax.rem(q, 4) * _D + d], v)
                return carry

            lax.fori_loop(0, 128 * grp // 16, tstep, 0)

        @pl.when(gid_of(0) < ng)
        def _():
            start_in(0, 0)

        def step(g, carry):
            for slot in range(2):
                k = g * 2 + slot
                gid = gid_of(k)

                @pl.when(gid < ng)
                def _():
                    @pl.when(gid_of(k + 1) < ng)
                    def _():
                        start_in(k + 1, 1 - slot)

                    for dt in range(_D // 8):
                        pltpu.make_async_copy(
                            tabt_hbm.at[pl.ds(dt * 8, 8),
                                        pl.ds(gid * 128 * grp, 128 * grp)],
                            in_v.at[slot, pl.ds(dt * 8, 8)],
                            gsems[slot]).wait()

                    @pl.when(k >= 2)
                    def _():
                        pltpu.make_async_copy(
                            ob_v.at[slot],
                            out_hbm.at[pl.ds(gid_of(k - 2) * 32 * grp,
                                             32 * grp)],
                            wsems[slot]).wait()

                    transp(slot)
                    pltpu.async_copy(
                        ob_v.at[slot],
                        out_hbm.at[pl.ds(gid * 32 * grp, 32 * grp)],
                        wsems[slot])
            return carry

        lax.fori_loop(0, (rounds + 1) // 2, step, 0)

        for s in range(2):
            kmax = 2 * ((rounds + 1) // 2)
            k1 = kmax - 1 if (kmax - 1) % 2 == s else kmax - 2
            kk = jnp.where(gid_of(k1) < ng, k1, k1 - 2)

            @pl.when(kk >= 0)
            def _():
                pltpu.make_async_copy(
                    ob_v.at[s],
                    out_hbm.at[pl.ds((kk * _NW + wid) * 32 * grp,
                                     32 * grp)],
                    wsems[s]).wait()

        if nlast:
            ncol = V - nfull * 128

            @pl.when(wid == 0)
            def _():
                for d in range(_D):
                    pltpu.async_copy(
                        tabt_hbm.at[d, pl.ds(nfull * 128, ncol)],
                        in_v.at[0, d, pl.ds(0, ncol)], gsems[0])
                for d in range(_D):
                    pltpu.make_async_copy(
                        tabt_hbm.at[d, pl.ds(nfull * 128, ncol)],
                        in_v.at[0, d, pl.ds(0, ncol)], gsems[0]).wait()
                ob = ob_v.at[0]
                for g in range(ncol // 16):
                    q = g * 16 + lane
                    for d in range(_D):
                        v = in_v[0, d, pl.ds(g * 16, 16)]
                        plsc.store_scatter(
                            ob, [q // 4, lax.rem(q, 4) * _D + d], v)
                pltpu.async_copy(
                    ob_v.at[0, pl.ds(0, nlast)],
                    out_hbm.at[pl.ds(nfull * 32, nlast)], wsems[0])
                pltpu.make_async_copy(
                    ob_v.at[0, pl.ds(0, nlast)],
                    out_hbm.at[pl.ds(nfull * 32, nlast)], wsems[0]).wait()

    return relayout_kernel


def _make_gather(B, H):
    assert B == _NW * _CH * H
    cpw = H
    assert cpw % _NBUF == 0

    @functools.partial(
        pl.kernel,
        out_type=jax.ShapeDtypeStruct((H, _D // 8, _NW, 1024), jnp.float32),
        mesh=plsc.VectorSubcoreMesh(core_axis_name="c", subcore_axis_name="s"),
        compiler_params=pltpu.CompilerParams(
            use_tc_tiling_on_sc=False, needs_layout_passes=False),
        scratch_types=(
            [pltpu.VMEM((_CH, H), jnp.int32),
             pltpu.VMEM((cpw, _CH), jnp.int32),
             pltpu.VMEM((_NBUF, _CH, _D), jnp.float32),
             pltpu.VMEM((_NBUF, _D * _CH), jnp.float32)]
            + [pltpu.SemaphoreType.DMA] * (2 * _NBUF)
        ),
    )
    def gather_kernel(x_hbm, tab_hbm, out_hbm, xblk_v, idx_v, rows_v,
                      patch_v, *sems):
        gsems = sems[:_NBUF]
        wsems = sems[_NBUF:]
        wid = lax.axis_index("s") * _NC + lax.axis_index("c")

        pltpu.sync_copy(x_hbm.at[pl.ds(wid * _CH, _CH)], xblk_v)

        lane = lax.iota(jnp.int32, 16)

        def transp(h, carry):
            hv = jnp.full((16,), h, jnp.int32)
            for l in range(_CH // 16):
                v = plsc.load_gather(xblk_v, [l * 16 + lane, hv])
                idx_v[h, pl.ds(l * 16, 16)] = v
            return carry

        lax.fori_loop(0, cpw, transp, 0)

        pos = [lane * 128 + c * 2048 for c in range(2)]

        for b in range(_NBUF):
            pltpu.async_copy(tab_hbm.at[idx_v.at[b]], rows_v.at[b], gsems[b])

        def step(g, carry):
            for b in range(_NBUF):
                j = g * _NBUF + b
                pltpu.make_async_copy(
                    tab_hbm.at[idx_v.at[j]], rows_v.at[b], gsems[b]).wait()

                @pl.when(g > 0)
                def _():
                    for i in range(_D // 8):
                        pltpu.make_async_copy(
                            patch_v.at[b, pl.ds(i * 1024, 1024)],
                            out_hbm.at[j - _NBUF, i, wid], wsems[b]).wait()

                patch = patch_v.at[b]
                for l in range(_CH):
                    for c in range(2):
                        v = rows_v[b, l, pl.ds(c * 16, 16)]
                        plsc.store_scatter(patch, [pos[c] + l], v)

                for i in range(_D // 8):
                    pltpu.async_copy(
                        patch_v.at[b, pl.ds(i * 1024, 1024)],
                        out_hbm.at[j, i, wid], wsems[b])

                nj = j + _NBUF

                @pl.when(nj < cpw)
                def _():
                    pltpu.async_copy(
                        tab_hbm.at[idx_v.at[nj]], rows_v.at[b], gsems[b])
            return carry

        lax.fori_loop(0, cpw // _NBUF, step, 0)

        for b in range(_NBUF):
            for i in range(_D // 8):
                pltpu.make_async_copy(
                    patch_v.at[b, pl.ds(i * 1024, 1024)],
                    out_hbm.at[cpw - _NBUF + b, i, wid], wsems[b]).wait()

    return gather_kernel


def kernel(x, table):
    batch, hist = x.shape
    B = batch * hist
    V = table.shape[0]
    rowmaj = _make_relayout(V)(jnp.swapaxes(table, 0, 1))
    out5 = _make_gather(B, hist)(x.astype(jnp.int32),
                                 rowmaj.reshape(V, _D))
    out6 = out5.reshape(hist, _D // 8, _NW, 8, 128)
    return lax.reshape(out6, (batch, hist, _D), dimensions=(2, 4, 0, 1, 3))


# speedup vs baseline: 1.1471x; 1.1471x over previous
import functools

import jax
import jax.numpy as jnp
from jax import lax
from jax.experimental import pallas as pl
from jax.experimental.pallas import tpu as pltpu
from jax.experimental.pallas import tpu_sc as plsc

_D = 32
_NC = 2
_NS = 16
_NW = _NC * _NS
_CH = 128
_NBUF = 4


def _make_gather(B, H):
    assert B % (_NW * _CH) == 0
    cpw = B // (_NW * _CH)
    nslab = B // _CH // H
    assert cpw % _NBUF == 0

    @functools.partial(
        pl.kernel,
        out_type=jax.ShapeDtypeStruct((H, _D // 8, nslab, 1024), jnp.float32),
        mesh=plsc.VectorSubcoreMesh(core_axis_name="c", subcore_axis_name="s"),
        compiler_params=pltpu.CompilerParams(
            use_tc_tiling_on_sc=False, needs_layout_passes=False),
        scratch_types=(
            [pltpu.VMEM((cpw, _CH), jnp.int32),
             pltpu.VMEM((_NBUF, _CH, _D), jnp.float32),
             pltpu.VMEM((_NBUF, _D * _CH), jnp.float32)]
            + [pltpu.SemaphoreType.DMA] * (2 * _NBUF)
        ),
    )
    def gather_kernel(x_hbm, tab_hbm, out_hbm, idx_v, rows_v, patch_v, *sems):
        gsems = sems[:_NBUF]
        wsems = sems[_NBUF:]
        wid = lax.axis_index("s") * _NC + lax.axis_index("c")
        rbase = wid * cpw

        pltpu.sync_copy(x_hbm.at[pl.ds(rbase, cpw)], idx_v)

        lane = lax.iota(jnp.int32, 16)
        pos = [lane * 128 + c * 2048 for c in range(2)]

        def out_tiles(j):
            gc = rbase + j
            return gc // nslab, gc % nslab

        for b in range(_NBUF):
            pltpu.async_copy(tab_hbm.at[idx_v.at[b]], rows_v.at[b], gsems[b])

        def step(g, carry):
            for b in range(_NBUF):
                j = g * _NBUF + b
                h, jb = out_tiles(j)
                pltpu.make_async_copy(
                    tab_hbm.at[idx_v.at[j]], rows_v.at[b], gsems[b]).wait()

                @pl.when(g > 0)
                def _():
                    hp, jp = out_tiles(j - _NBUF)
                    for i in range(_D // 8):
                        pltpu.make_async_copy(
                            patch_v.at[b, pl.ds(i * 1024, 1024)],
                            out_hbm.at[hp, i, jp], wsems[b]).wait()

                patch = patch_v.at[b]
                for l in range(_CH):
                    for c in range(2):
                        v = rows_v[b, l, pl.ds(c * 16, 16)]
                        plsc.store_scatter(patch, [pos[c] + l], v)

                for i in range(_D // 8):
                    pltpu.async_copy(
                        patch_v.at[b, pl.ds(i * 1024, 1024)],
                        out_hbm.at[h, i, jb], wsems[b])

                nj = j + _NBUF

                @pl.when(nj < cpw)
                def _():
                    pltpu.async_copy(
                        tab_hbm.at[idx_v.at[nj]], rows_v.at[b], gsems[b])
            return carry

        lax.fori_loop(0, cpw // _NBUF, step, 0)

        for b in range(_NBUF):
            h, jb = out_tiles(cpw - _NBUF + b)
            for i in range(_D // 8):
                pltpu.make_async_copy(
                    patch_v.at[b, pl.ds(i * 1024, 1024)],
                    out_hbm.at[h, i, jb], wsems[b]).wait()

    return gather_kernel


def kernel(x, table):
    batch, hist = x.shape
    B = batch * hist
    xf = x.astype(jnp.int32).T.reshape(B // _CH, _CH)
    out5 = _make_gather(B, hist)(xf, table)
    out6 = out5.reshape(hist, _D // 8, B // _CH // hist, 8, 128)
    return out6.transpose(2, 4, 0, 1, 3).reshape(batch, hist, _D)
